# baseline (speedup 1.0000x reference)
import jax
import jax.numpy as jnp
from jax.experimental import pallas as pl
from jax.experimental.pallas import tpu as pltpu

_SLOPE = 0.1
_ST = 80
_B = 96
_L = 64 * _ST
_SZ = 5312


def _leaky(x):
    return jnp.maximum(x, _SLOPE * x)


def _conv3(src_ref, wp, bias_row):
    xv = src_ref[...]
    views = [xv[15 + dx: 15 + dx + _L + 160] for dx in range(3)]
    slabs = [views[k % 3][(k // 3) * _ST: (k // 3) * _ST + _L] for k in range(9)]
    lhs = jnp.concatenate(slabs, axis=1)
    acc = jnp.dot(lhs, wp, preferred_element_type=jnp.float32)
    return acc + bias_row


def _body(x1_ref, x2_ref, w1_ref, w2_ref, wd1_ref, wp2_ref, w3_ref,
          wp4_ref, w5_ref, b1_ref, b2_ref, bd1_ref, bd2_ref, bd3_ref,
          bd4_ref, bd5_ref, o_ref, d1p, d3p):
    f32, bf16 = jnp.float32, jnp.bfloat16
    b1, b2, bd1 = b1_ref[...], b2_ref[...], bd1_ref[...]
    bd2, bd3, bd4 = bd2_ref[...], bd3_ref[...], bd4_ref[...]

    w1 = w1_ref[...].astype(bf16)
    w2 = w2_ref[...].astype(bf16)
    wd1 = wd1_ref[...].astype(bf16)
    wd1a, wd1b = wd1[:, :128], wd1[:, 128:]
    w3 = w3_ref[...].astype(bf16)
    w5 = w5_ref[...].astype(bf16)

    dn_tt = (((0,), (1,)), ((), ()))
    dn_nt = (((1,), (1,)), ((), ()))

    x1v = x1_ref[...].reshape(256, 1024).astype(bf16)
    y1 = jax.lax.dot_general(x1v, w1, dimension_numbers=dn_tt,
                             preferred_element_type=f32)
    y1 = _leaky(y1 + b1).astype(bf16)
    x2v = x2_ref[...].reshape(256, 4096).astype(bf16)
    y2 = jax.lax.dot_general(x2v, w2, dimension_numbers=dn_tt,
                             preferred_element_type=f32)
    y2 = _leaky(y2 + b2).astype(bf16)

    z1 = jax.lax.dot_general(y1, wd1a, dimension_numbers=dn_nt,
                             preferred_element_type=f32)
    z2 = jax.lax.dot_general(y2, wd1b, dimension_numbers=dn_nt,
                             preferred_element_type=f32) + bd1

    ii = jax.lax.broadcasted_iota(jnp.int32, (64, 32), 0)
    kk = jax.lax.broadcasted_iota(jnp.int32, (64, 32), 1)
    U = ((ii // 2) == kk).astype(f32)

    d1p[...] = jnp.zeros((_SZ, 128), bf16)
    zgap = jnp.zeros((16, 128), bf16)
    for h in range(32):
        zrow = z1[32 * h: 32 * h + 32]
        zup = jnp.dot(U, zrow, preferred_element_type=f32)
        r0 = _leaky(zup + z2[128 * h: 128 * h + 64]).astype(bf16)
        r1 = _leaky(zup + z2[128 * h + 64: 128 * h + 128]).astype(bf16)
        d1p[_B + 160 * h: _B + 160 * h + 160, :] = jnp.concatenate(
            [r0, zgap, r1, zgap], axis=0)

    h2 = _leaky(_conv3(d1p, wp2_ref[...], bd2)).astype(bf16)
    d3 = _leaky(jax.lax.dot_general(h2, w3, dimension_numbers=dn_nt,
                                    preferred_element_type=f32) + bd3)
    qi = jax.lax.broadcasted_iota(jnp.int32, (_L, 1), 0)
    gmask = ((qi % _ST) < 64).astype(f32)
    d3p[...] = jnp.zeros((_SZ, 128), bf16)
    d3p[_B: _B + _L, :] = (d3 * gmask).astype(bf16)

    h4 = _leaky(_conv3(d3p, wp4_ref[...], bd4)).astype(bf16)
    h4c = jnp.concatenate([h4[_ST * j: _ST * j + 64] for j in range(64)], axis=0)
    outv = jax.lax.dot_general(w5, h4c, dimension_numbers=dn_nt,
                               preferred_element_type=f32)
    o_ref[...] = _leaky(outv + bd5_ref[...])


def _pack_taps(w_oihw):
    return jnp.transpose(w_oihw, (2, 3, 1, 0)).reshape(9 * 128, 256).astype(jnp.bfloat16)


def kernel(x1, x2, conv1x1_1_w, conv1x1_1_b, conv1x1_2_w, conv1x1_2_b,
           dbl1_w, dbl1_b, dbl2_w, dbl2_b, dbl3_w, dbl3_b, dbl4_w, dbl4_b,
           dbl5_w, dbl5_b):
    n, c, h, w = x1.shape
    c2 = c // 2
    h2, w2 = 2 * h, 2 * w
    bf16 = jnp.bfloat16

    wp2 = _pack_taps(dbl2_w)
    wp4 = _pack_taps(dbl4_w)

    half = n // 2
    out = pl.pallas_call(
        _body,
        out_shape=jax.ShapeDtypeStruct((n, c2, h2 * w2), jnp.float32),
        grid=(2, half),
        in_specs=[
            pl.BlockSpec((None, c, h, w), lambda cc, i: (cc * half + i, 0, 0, 0)),
            pl.BlockSpec((None, c, h2, w2),
                         lambda cc, i: (cc * half + i, 0, 0, 0)),
            pl.BlockSpec((c2, c), lambda cc, i: (0, 0)),
            pl.BlockSpec((c2, c), lambda cc, i: (0, 0)),
            pl.BlockSpec((c2, c), lambda cc, i: (0, 0)),
            pl.BlockSpec((9 * c2, c), lambda cc, i: (0, 0)),
            pl.BlockSpec((c2, c), lambda cc, i: (0, 0)),
            pl.BlockSpec((9 * c2, c), lambda cc, i: (0, 0)),
            pl.BlockSpec((c2, c), lambda cc, i: (0, 0)),
            pl.BlockSpec((1, c2), lambda cc, i: (0, 0)),
            pl.BlockSpec((1, c2), lambda cc, i: (0, 0)),
            pl.BlockSpec((1, c2), lambda cc, i: (0, 0)),
            pl.BlockSpec((1, c), lambda cc, i: (0, 0)),
            pl.BlockSpec((1, c2), lambda cc, i: (0, 0)),
            pl.BlockSpec((1, c), lambda cc, i: (0, 0)),
            pl.BlockSpec((c2, 1), lambda cc, i: (0, 0)),
        ],
        out_specs=pl.BlockSpec((None, c2, h2 * w2),
                               lambda cc, i: (cc * half + i, 0, 0)),
        scratch_shapes=[
            pltpu.VMEM((_SZ, c2), bf16),
            pltpu.VMEM((_SZ, c2), bf16),
        ],
        compiler_params=pltpu.CompilerParams(
            dimension_semantics=("parallel", "arbitrary"),
            vmem_limit_bytes=60 * 1024 * 1024,
        ),
    )(x1, x2,
      conv1x1_1_w.reshape(c2, c), conv1x1_2_w.reshape(c2, c),
      dbl1_w.reshape(c2, c), wp2, dbl3_w.reshape(c2, c), wp4,
      dbl5_w.reshape(c2, c),
      conv1x1_1_b.reshape(1, c2), conv1x1_2_b.reshape(1, c2),
      dbl1_b.reshape(1, c2), dbl2_b.reshape(1, c), dbl3_b.reshape(1, c2),
      dbl4_b.reshape(1, c), dbl5_b.reshape(c2, 1))

    return out.reshape(n, c2, h2, w2)


# speedup vs baseline: 1.3423x; 1.3423x over previous
import jax
import jax.numpy as jnp
from jax.experimental import pallas as pl
from jax.experimental.pallas import tpu as pltpu

_SLOPE = 0.1
_ST = 80
_B = 96
_L = 64 * _ST
_SZ = 5312


def _leaky(x):
    return jnp.maximum(x, _SLOPE * x)


def _conv3(src_ref, wp, bias_row):
    xv = src_ref[...]
    views = [xv[15 + dx: 15 + dx + _L + 160] for dx in range(3)]
    slabs = [views[k % 3][(k // 3) * _ST: (k // 3) * _ST + _L] for k in range(9)]
    lhs = jnp.concatenate(slabs, axis=1)
    acc = jnp.dot(lhs, wp, preferred_element_type=jnp.float32)
    return acc + bias_row


def _body(x1_ref, x2_ref, w1_ref, w2_ref, wd1_ref, wp2_ref, w3_ref,
          wp4_ref, w5_ref, b1_ref, b2_ref, bd1_ref, bd2_ref, bd3_ref,
          bd4_ref, bd5_ref, o_ref, d1p, d3p):
    f32, bf16 = jnp.float32, jnp.bfloat16
    b1, b2, bd1 = b1_ref[...], b2_ref[...], bd1_ref[...]
    bd2, bd3, bd4 = bd2_ref[...], bd3_ref[...], bd4_ref[...]

    w1 = w1_ref[...].astype(bf16)
    w2 = w2_ref[...].astype(bf16)
    wd1 = wd1_ref[...].astype(bf16)
    wd1a, wd1b = wd1[:, :128], wd1[:, 128:]
    w3 = w3_ref[...].astype(bf16)
    w5 = w5_ref[...].astype(bf16)

    dn_tt = (((0,), (1,)), ((), ()))
    dn_nt = (((1,), (1,)), ((), ()))

    y1 = jax.lax.dot_general(x1_ref[...], w1, dimension_numbers=dn_tt,
                             preferred_element_type=f32)
    y1 = _leaky(y1 + b1).astype(bf16)
    y2 = jax.lax.dot_general(x2_ref[...], w2, dimension_numbers=dn_tt,
                             preferred_element_type=f32)
    y2 = _leaky(y2 + b2).astype(bf16)

    z1 = jax.lax.dot_general(y1, wd1a, dimension_numbers=dn_nt,
                             preferred_element_type=f32)
    z2 = jax.lax.dot_general(y2, wd1b, dimension_numbers=dn_nt,
                             preferred_element_type=f32) + bd1

    ii = jax.lax.broadcasted_iota(jnp.int32, (64, 32), 0)
    kk = jax.lax.broadcasted_iota(jnp.int32, (64, 32), 1)
    U = ((ii // 2) == kk).astype(f32)

    d1p[...] = jnp.zeros((_SZ, 128), bf16)
    zgap = jnp.zeros((16, 128), bf16)
    for h in range(32):
        zrow = z1[32 * h: 32 * h + 32]
        zup = jnp.dot(U, zrow, preferred_element_type=f32)
        r0 = _leaky(zup + z2[128 * h: 128 * h + 64]).astype(bf16)
        r1 = _leaky(zup + z2[128 * h + 64: 128 * h + 128]).astype(bf16)
        d1p[_B + 160 * h: _B + 160 * h + 160, :] = jnp.concatenate(
            [r0, zgap, r1, zgap], axis=0)

    h2 = _leaky(_conv3(d1p, wp2_ref[...], bd2)).astype(bf16)
    d3 = _leaky(jax.lax.dot_general(h2, w3, dimension_numbers=dn_nt,
                                    preferred_element_type=f32) + bd3)
    qi = jax.lax.broadcasted_iota(jnp.int32, (_L, 1), 0)
    gmask = ((qi % _ST) < 64).astype(f32)
    d3p[...] = jnp.zeros((_SZ, 128), bf16)
    d3p[_B: _B + _L, :] = (d3 * gmask).astype(bf16)

    h4 = _leaky(_conv3(d3p, wp4_ref[...], bd4)).astype(bf16)
    h4c = jnp.concatenate([h4[_ST * j: _ST * j + 64] for j in range(64)], axis=0)
    outv = jax.lax.dot_general(w5, h4c, dimension_numbers=dn_nt,
                               preferred_element_type=f32)
    o_ref[...] = _leaky(outv + bd5_ref[...])


def _pack_taps(w_oihw):
    return jnp.transpose(w_oihw, (2, 3, 1, 0)).reshape(9 * 128, 256).astype(jnp.bfloat16)


def kernel(x1, x2, conv1x1_1_w, conv1x1_1_b, conv1x1_2_w, conv1x1_2_b,
           dbl1_w, dbl1_b, dbl2_w, dbl2_b, dbl3_w, dbl3_b, dbl4_w, dbl4_b,
           dbl5_w, dbl5_b):
    n, c, h, w = x1.shape
    c2 = c // 2
    h2, w2 = 2 * h, 2 * w
    bf16 = jnp.bfloat16

    x1s = x1.reshape(n, c, h * w).astype(bf16)
    x2s = x2.reshape(n, c, h2 * w2).astype(bf16)
    wp2 = _pack_taps(dbl2_w)
    wp4 = _pack_taps(dbl4_w)

    half = n // 2
    out = pl.pallas_call(
        _body,
        out_shape=jax.ShapeDtypeStruct((n, c2, h2 * w2), jnp.float32),
        grid=(2, half),
        in_specs=[
            pl.BlockSpec((None, c, h * w), lambda cc, i: (cc * half + i, 0, 0)),
            pl.BlockSpec((None, c, h2 * w2), lambda cc, i: (cc * half + i, 0, 0)),
            pl.BlockSpec((c2, c), lambda cc, i: (0, 0)),
            pl.BlockSpec((c2, c), lambda cc, i: (0, 0)),
            pl.BlockSpec((c2, c), lambda cc, i: (0, 0)),
            pl.BlockSpec((9 * c2, c), lambda cc, i: (0, 0)),
            pl.BlockSpec((c2, c), lambda cc, i: (0, 0)),
            pl.BlockSpec((9 * c2, c), lambda cc, i: (0, 0)),
            pl.BlockSpec((c2, c), lambda cc, i: (0, 0)),
            pl.BlockSpec((1, c2), lambda cc, i: (0, 0)),
            pl.BlockSpec((1, c2), lambda cc, i: (0, 0)),
            pl.BlockSpec((1, c2), lambda cc, i: (0, 0)),
            pl.BlockSpec((1, c), lambda cc, i: (0, 0)),
            pl.BlockSpec((1, c2), lambda cc, i: (0, 0)),
            pl.BlockSpec((1, c), lambda cc, i: (0, 0)),
            pl.BlockSpec((c2, 1), lambda cc, i: (0, 0)),
        ],
        out_specs=pl.BlockSpec((None, c2, h2 * w2),
                               lambda cc, i: (cc * half + i, 0, 0)),
        scratch_shapes=[
            pltpu.VMEM((_SZ, c2), bf16),
            pltpu.VMEM((_SZ, c2), bf16),
        ],
        compiler_params=pltpu.CompilerParams(
            dimension_semantics=("parallel", "arbitrary"),
            vmem_limit_bytes=60 * 1024 * 1024,
        ),
    )(x1s, x2s,
      conv1x1_1_w.reshape(c2, c), conv1x1_2_w.reshape(c2, c),
      dbl1_w.reshape(c2, c), wp2, dbl3_w.reshape(c2, c), wp4,
      dbl5_w.reshape(c2, c),
      conv1x1_1_b.reshape(1, c2), conv1x1_2_b.reshape(1, c2),
      dbl1_b.reshape(1, c2), dbl2_b.reshape(1, c), dbl3_b.reshape(1, c2),
      dbl4_b.reshape(1, c), dbl5_b.reshape(c2, 1))

    return out.reshape(n, c2, h2, w2)


# speedup vs baseline: 1.3821x; 1.0296x over previous
import jax
import jax.numpy as jnp
from jax.experimental import pallas as pl
from jax.experimental.pallas import tpu as pltpu

_SLOPE = 0.1
_ST = 80
_B = 96
_L = 64 * _ST
_SZ = 5312


def _leaky(x):
    return jnp.maximum(x, _SLOPE * x)


def _conv3(src_ref, wp, bias_row, stg):
    xv = src_ref[...]
    stg[0, :, :] = xv[15: 15 + _L + 160]
    stg[1, :, :] = xv[17: 17 + _L + 160]
    views = [stg[0, :, :], xv[16: 16 + _L + 160], stg[1, :, :]]
    slabs = [views[k % 3][(k // 3) * _ST: (k // 3) * _ST + _L] for k in range(9)]
    lhs = jnp.concatenate(slabs, axis=1)
    acc = jnp.dot(lhs, wp, preferred_element_type=jnp.float32)
    return acc + bias_row


def _body(x1_ref, x2_ref, w1_ref, w2_ref, wd1_ref, wp2_ref, w3_ref,
          wp4_ref, w5_ref, b1_ref, b2_ref, bd1_ref, bd2_ref, bd3_ref,
          bd4_ref, bd5_ref, o_ref, d1p, d3p, stg):
    f32, bf16 = jnp.float32, jnp.bfloat16
    b1, b2, bd1 = b1_ref[...], b2_ref[...], bd1_ref[...]
    bd2, bd3, bd4 = bd2_ref[...], bd3_ref[...], bd4_ref[...]

    w1 = w1_ref[...].astype(bf16)
    w2 = w2_ref[...].astype(bf16)
    wd1 = wd1_ref[...].astype(bf16)
    wd1a, wd1b = wd1[:, :128], wd1[:, 128:]
    w3 = w3_ref[...].astype(bf16)
    w5 = w5_ref[...].astype(bf16)

    dn_tt = (((0,), (1,)), ((), ()))
    dn_nt = (((1,), (1,)), ((), ()))

    y1 = jax.lax.dot_general(x1_ref[...].astype(bf16), w1,
                             dimension_numbers=dn_tt,
                             preferred_element_type=f32)
    y1 = _leaky(y1 + b1).astype(bf16)
    y2 = jax.lax.dot_general(x2_ref[...].astype(bf16), w2,
                             dimension_numbers=dn_tt,
                             preferred_element_type=f32)
    y2 = _leaky(y2 + b2).astype(bf16)

    z1 = jax.lax.dot_general(y1, wd1a, dimension_numbers=dn_nt,
                             preferred_element_type=f32)
    z2 = jax.lax.dot_general(y2, wd1b, dimension_numbers=dn_nt,
                             preferred_element_type=f32) + bd1

    ii = jax.lax.broadcasted_iota(jnp.int32, (64, 32), 0)
    kk = jax.lax.broadcasted_iota(jnp.int32, (64, 32), 1)
    U = ((ii // 2) == kk).astype(f32)

    d1p[...] = jnp.zeros((_SZ, 128), bf16)
    zgap = jnp.zeros((16, 128), bf16)
    for h in range(32):
        zrow = z1[32 * h: 32 * h + 32]
        zup = jnp.dot(U, zrow, preferred_element_type=f32)
        r0 = _leaky(zup + z2[128 * h: 128 * h + 64]).astype(bf16)
        r1 = _leaky(zup + z2[128 * h + 64: 128 * h + 128]).astype(bf16)
        d1p[_B + 160 * h: _B + 160 * h + 160, :] = jnp.concatenate(
            [r0, zgap, r1, zgap], axis=0)

    h2 = _leaky(_conv3(d1p, wp2_ref[...], bd2, stg)).astype(bf16)
    d3 = _leaky(jax.lax.dot_general(h2, w3, dimension_numbers=dn_nt,
                                    preferred_element_type=f32) + bd3)
    qi = jax.lax.broadcasted_iota(jnp.int32, (_L, 1), 0)
    gmask = ((qi % _ST) < 64).astype(f32)
    d3p[...] = jnp.zeros((_SZ, 128), bf16)
    d3p[_B: _B + _L, :] = (d3 * gmask).astype(bf16)

    h4 = _leaky(_conv3(d3p, wp4_ref[...], bd4, stg)).astype(bf16)
    h4c = jnp.concatenate([h4[_ST * j: _ST * j + 64] for j in range(64)], axis=0)
    outv = jax.lax.dot_general(w5, h4c, dimension_numbers=dn_nt,
                               preferred_element_type=f32)
    o_ref[...] = _leaky(outv + bd5_ref[...])


def _pack_taps(w_oihw):
    return jnp.transpose(w_oihw, (2, 3, 1, 0)).reshape(9 * 128, 256).astype(jnp.bfloat16)


def kernel(x1, x2, conv1x1_1_w, conv1x1_1_b, conv1x1_2_w, conv1x1_2_b,
           dbl1_w, dbl1_b, dbl2_w, dbl2_b, dbl3_w, dbl3_b, dbl4_w, dbl4_b,
           dbl5_w, dbl5_b):
    n, c, h, w = x1.shape
    c2 = c // 2
    h2, w2 = 2 * h, 2 * w
    bf16 = jnp.bfloat16

    x1s = x1.reshape(n, c, h * w)
    x2s = x2.reshape(n, c, h2 * w2)
    wp2 = _pack_taps(dbl2_w)
    wp4 = _pack_taps(dbl4_w)

    half = n // 2
    out = pl.pallas_call(
        _body,
        out_shape=jax.ShapeDtypeStruct((n, c2, h2 * w2), jnp.float32),
        grid=(2, half),
        in_specs=[
            pl.BlockSpec((None, c, h * w), lambda cc, i: (cc * half + i, 0, 0)),
            pl.BlockSpec((None, c, h2 * w2), lambda cc, i: (cc * half + i, 0, 0)),
            pl.BlockSpec((c2, c), lambda cc, i: (0, 0)),
            pl.BlockSpec((c2, c), lambda cc, i: (0, 0)),
            pl.BlockSpec((c2, c), lambda cc, i: (0, 0)),
            pl.BlockSpec((9 * c2, c), lambda cc, i: (0, 0)),
            pl.BlockSpec((c2, c), lambda cc, i: (0, 0)),
            pl.BlockSpec((9 * c2, c), lambda cc, i: (0, 0)),
            pl.BlockSpec((c2, c), lambda cc, i: (0, 0)),
            pl.BlockSpec((1, c2), lambda cc, i: (0, 0)),
            pl.BlockSpec((1, c2), lambda cc, i: (0, 0)),
            pl.BlockSpec((1, c2), lambda cc, i: (0, 0)),
            pl.BlockSpec((1, c), lambda cc, i: (0, 0)),
            pl.BlockSpec((1, c2), lambda cc, i: (0, 0)),
            pl.BlockSpec((1, c), lambda cc, i: (0, 0)),
            pl.BlockSpec((c2, 1), lambda cc, i: (0, 0)),
        ],
        out_specs=pl.BlockSpec((None, c2, h2 * w2),
                               lambda cc, i: (cc * half + i, 0, 0)),
        scratch_shapes=[
            pltpu.VMEM((_SZ, c2), bf16),
            pltpu.VMEM((_SZ, c2), bf16),
            pltpu.VMEM((2, _L + 160, c2), bf16),
        ],
        compiler_params=pltpu.CompilerParams(
            dimension_semantics=("parallel", "arbitrary"),
            vmem_limit_bytes=60 * 1024 * 1024,
        ),
    )(x1s, x2s,
      conv1x1_1_w.reshape(c2, c), conv1x1_2_w.reshape(c2, c),
      dbl1_w.reshape(c2, c), wp2, dbl3_w.reshape(c2, c), wp4,
      dbl5_w.reshape(c2, c),
      conv1x1_1_b.reshape(1, c2), conv1x1_2_b.reshape(1, c2),
      dbl1_b.reshape(1, c2), dbl2_b.reshape(1, c), dbl3_b.reshape(1, c2),
      dbl4_b.reshape(1, c), dbl5_b.reshape(c2, 1))

    return out.reshape(n, c2, h2, w2)


# speedup vs baseline: 1.3930x; 1.0079x over previous
import jax
import jax.numpy as jnp
from jax.experimental import pallas as pl
from jax.experimental.pallas import tpu as pltpu

_SLOPE = 0.1
_ST = 80
_B = 96
_L = 64 * _ST
_SZ = 5312


def _leaky(x):
    return jnp.maximum(x, _SLOPE * x)


def _conv3(src_ref, wp, bias_row):
    xv = src_ref[...]
    views = [xv[15 + dx: 15 + dx + _L + 160] for dx in range(3)]
    slabs = [views[k % 3][(k // 3) * _ST: (k // 3) * _ST + _L] for k in range(9)]
    lhs = jnp.concatenate(slabs, axis=1)
    acc = jnp.dot(lhs, wp, preferred_element_type=jnp.float32)
    return acc + bias_row


def _body(x1_ref, x2_ref, w1_ref, w2_ref, wd1_ref, wp2_ref, w3_ref,
          wp4_ref, w5_ref, b1_ref, b2_ref, bd1_ref, bd2_ref, bd3_ref,
          bd4_ref, bd5_ref, o_ref, d1p, d3p):
    f32, bf16 = jnp.float32, jnp.bfloat16
    b1, b2, bd1 = b1_ref[...], b2_ref[...], bd1_ref[...]
    bd2, bd3, bd4 = bd2_ref[...], bd3_ref[...], bd4_ref[...]

    w1 = w1_ref[...].astype(bf16)
    w2 = w2_ref[...].astype(bf16)
    wd1 = wd1_ref[...].astype(bf16)
    wd1a, wd1b = wd1[:, :128], wd1[:, 128:]
    w3 = w3_ref[...].astype(bf16)
    w5 = w5_ref[...].astype(bf16)

    dn_tt = (((0,), (1,)), ((), ()))
    dn_nt = (((1,), (1,)), ((), ()))

    y1 = jax.lax.dot_general(x1_ref[...].astype(bf16), w1,
                             dimension_numbers=dn_tt,
                             preferred_element_type=f32)
    y1 = _leaky(y1 + b1).astype(bf16)
    y2 = jax.lax.dot_general(x2_ref[...].astype(bf16), w2,
                             dimension_numbers=dn_tt,
                             preferred_element_type=f32)
    y2 = _leaky(y2 + b2).astype(bf16)

    z1 = jax.lax.dot_general(y1, wd1a, dimension_numbers=dn_nt,
                             preferred_element_type=f32)
    z2 = jax.lax.dot_general(y2, wd1b, dimension_numbers=dn_nt,
                             preferred_element_type=f32) + bd1

    ii = jax.lax.broadcasted_iota(jnp.int32, (64, 32), 0)
    kk = jax.lax.broadcasted_iota(jnp.int32, (64, 32), 1)
    U = ((ii // 2) == kk).astype(f32)

    d1p[...] = jnp.zeros((_SZ, 128), bf16)
    for h in range(32):
        zrow = z1[32 * h: 32 * h + 32]
        zup = jnp.dot(U, zrow, preferred_element_type=f32)
        r0 = _leaky(zup + z2[128 * h: 128 * h + 64]).astype(bf16)
        r1 = _leaky(zup + z2[128 * h + 64: 128 * h + 128]).astype(bf16)
        d1p[_B + 160 * h: _B + 160 * h + 64, :] = r0
        d1p[_B + 160 * h + 80: _B + 160 * h + 144, :] = r1

    h2 = _leaky(_conv3(d1p, wp2_ref[...], bd2)).astype(bf16)
    d3 = _leaky(jax.lax.dot_general(h2, w3, dimension_numbers=dn_nt,
                                    preferred_element_type=f32) + bd3)
    qi = jax.lax.broadcasted_iota(jnp.int32, (_L, 1), 0)
    gmask = ((qi % _ST) < 64).astype(f32)
    d3p[...] = jnp.zeros((_SZ, 128), bf16)
    d3p[_B: _B + _L, :] = (d3 * gmask).astype(bf16)

    h4 = _leaky(_conv3(d3p, wp4_ref[...], bd4)).astype(bf16)
    h4c = jnp.concatenate([h4[_ST * j: _ST * j + 64] for j in range(64)], axis=0)
    outv = jax.lax.dot_general(w5, h4c, dimension_numbers=dn_nt,
                               preferred_element_type=f32)
    o_ref[...] = _leaky(outv + bd5_ref[...])


def _pack_taps(w_oihw):
    return jnp.transpose(w_oihw, (2, 3, 1, 0)).reshape(9 * 128, 256).astype(jnp.bfloat16)


def kernel(x1, x2, conv1x1_1_w, conv1x1_1_b, conv1x1_2_w, conv1x1_2_b,
           dbl1_w, dbl1_b, dbl2_w, dbl2_b, dbl3_w, dbl3_b, dbl4_w, dbl4_b,
           dbl5_w, dbl5_b):
    n, c, h, w = x1.shape
    c2 = c // 2
    h2, w2 = 2 * h, 2 * w
    bf16 = jnp.bfloat16

    x1s = x1.reshape(n, c, h * w)
    x2s = x2.reshape(n, c, h2 * w2)
    wp2 = _pack_taps(dbl2_w)
    wp4 = _pack_taps(dbl4_w)

    half = n // 2
    out = pl.pallas_call(
        _body,
        out_shape=jax.ShapeDtypeStruct((n, c2, h2 * w2), jnp.float32),
        grid=(2, half),
        in_specs=[
            pl.BlockSpec((None, c, h * w), lambda cc, i: (cc * half + i, 0, 0)),
            pl.BlockSpec((None, c, h2 * w2), lambda cc, i: (cc * half + i, 0, 0)),
            pl.BlockSpec((c2, c), lambda cc, i: (0, 0)),
            pl.BlockSpec((c2, c), lambda cc, i: (0, 0)),
            pl.BlockSpec((c2, c), lambda cc, i: (0, 0)),
            pl.BlockSpec((9 * c2, c), lambda cc, i: (0, 0)),
            pl.BlockSpec((c2, c), lambda cc, i: (0, 0)),
            pl.BlockSpec((9 * c2, c), lambda cc, i: (0, 0)),
            pl.BlockSpec((c2, c), lambda cc, i: (0, 0)),
            pl.BlockSpec((1, c2), lambda cc, i: (0, 0)),
            pl.BlockSpec((1, c2), lambda cc, i: (0, 0)),
            pl.BlockSpec((1, c2), lambda cc, i: (0, 0)),
            pl.BlockSpec((1, c), lambda cc, i: (0, 0)),
            pl.BlockSpec((1, c2), lambda cc, i: (0, 0)),
            pl.BlockSpec((1, c), lambda cc, i: (0, 0)),
            pl.BlockSpec((c2, 1), lambda cc, i: (0, 0)),
        ],
        out_specs=pl.BlockSpec((None, c2, h2 * w2),
                               lambda cc, i: (cc * half + i, 0, 0)),
        scratch_shapes=[
            pltpu.VMEM((_SZ, c2), bf16),
            pltpu.VMEM((_SZ, c2), bf16),
        ],
        compiler_params=pltpu.CompilerParams(
            dimension_semantics=("parallel", "arbitrary"),
            vmem_limit_bytes=60 * 1024 * 1024,
        ),
    )(x1s, x2s,
      conv1x1_1_w.reshape(c2, c), conv1x1_2_w.reshape(c2, c),
      dbl1_w.reshape(c2, c), wp2, dbl3_w.reshape(c2, c), wp4,
      dbl5_w.reshape(c2, c),
      conv1x1_1_b.reshape(1, c2), conv1x1_2_b.reshape(1, c2),
      dbl1_b.reshape(1, c2), dbl2_b.reshape(1, c), dbl3_b.reshape(1, c2),
      dbl4_b.reshape(1, c), dbl5_b.reshape(c2, 1))

    return out.reshape(n, c2, h2, w2)
